# baseline (speedup 1.0000x reference)
import jax
import jax.numpy as jnp
from jax.experimental import pallas as pl
from jax.experimental.pallas import tpu as pltpu

_BX = 512
_CH = 512


def _fused_kernel(x_ref, y_ref, wq_ref, bq_ref, wk_ref, bk_ref,
                  wv_ref, bv_ref, o_ref, k_sc, v_sc):
    i = pl.program_id(1)

    @pl.when(i == 0)
    def _project_kv():
        y = y_ref[0].astype(jnp.bfloat16)
        k = jax.lax.dot_general(y, wk_ref[...],
                                (((1,), (0,)), ((), ())),
                                preferred_element_type=jnp.float32)
        k_sc[...] = (k + bk_ref[...]).astype(jnp.bfloat16)
        v = jax.lax.dot_general(y, wv_ref[...],
                                (((1,), (0,)), ((), ())),
                                preferred_element_type=jnp.float32)
        v_sc[...] = (v + bv_ref[...]).astype(jnp.bfloat16)

    x = x_ref[0]
    q = jax.lax.dot_general(x.astype(jnp.bfloat16), wq_ref[...],
                            (((1,), (0,)), ((), ())),
                            preferred_element_type=jnp.float32)
    q = (q + bq_ref[...]).astype(jnp.bfloat16)
    nch = v_sc.shape[0] // _CH
    o = None
    for j in range(nch):
        sj = jax.lax.dot_general(q, k_sc[j * _CH:(j + 1) * _CH, :],
                                 (((1,), (1,)), ((), ())),
                                 preferred_element_type=jnp.float32)
        mj = jnp.max(sj, axis=-1, keepdims=True)
        if o is None:
            m = mj
            ej = jnp.exp(sj - m)
            l = jnp.sum(ej, axis=-1, keepdims=True)
            o = jax.lax.dot_general(ej.astype(jnp.bfloat16),
                                    v_sc[j * _CH:(j + 1) * _CH, :],
                                    (((1,), (0,)), ((), ())),
                                    preferred_element_type=jnp.float32)
        else:
            mn = jnp.maximum(m, mj)
            c = jnp.exp(m - mn)
            ej = jnp.exp(sj - mn)
            oj = jax.lax.dot_general(ej.astype(jnp.bfloat16),
                                     v_sc[j * _CH:(j + 1) * _CH, :],
                                     (((1,), (0,)), ((), ())),
                                     preferred_element_type=jnp.float32)
            o = c * o + oj
            l = c * l + jnp.sum(ej, axis=-1, keepdims=True)
            m = mn
    o_ref[0] = o * (1.0 / l) + x


def kernel(x, y, Wq, bq, Wk, bk, Wv, bv):
    B, SX, D = x.shape
    SY = y.shape[1]
    bq2 = bq.reshape(1, D)
    bk2 = bk.reshape(1, D)
    bv2 = bv.reshape(1, D)

    return pl.pallas_call(
        _fused_kernel,
        grid=(B, SX // _BX),
        in_specs=[
            pl.BlockSpec((1, _BX, D), lambda b, i: (b, i, 0)),
            pl.BlockSpec((1, SY, D), lambda b, i: (b, 0, 0)),
            pl.BlockSpec((D, D), lambda b, i: (0, 0)),
            pl.BlockSpec((1, D), lambda b, i: (0, 0)),
            pl.BlockSpec((D, D), lambda b, i: (0, 0)),
            pl.BlockSpec((1, D), lambda b, i: (0, 0)),
            pl.BlockSpec((D, D), lambda b, i: (0, 0)),
            pl.BlockSpec((1, D), lambda b, i: (0, 0)),
        ],
        out_specs=pl.BlockSpec((1, _BX, D), lambda b, i: (b, i, 0)),
        out_shape=jax.ShapeDtypeStruct((B, SX, D), jnp.float32),
        scratch_shapes=[
            pltpu.VMEM((SY, D), jnp.bfloat16),
            pltpu.VMEM((SY, D), jnp.bfloat16),
        ],
    )(x, y, Wq.astype(jnp.bfloat16), bq2, Wk.astype(jnp.bfloat16), bk2,
      Wv.astype(jnp.bfloat16), bv2)


# speedup vs baseline: 1.1611x; 1.1611x over previous
import jax
import jax.numpy as jnp
from jax.experimental import pallas as pl
from jax.experimental.pallas import tpu as pltpu

_BX = 512
_CH = 512


def _fused_kernel(x_ref, y_ref, wq_ref, bq_ref, wk_ref, bk_ref,
                  wv_ref, bv_ref, o_ref, k_sc, v_sc):
    i = pl.program_id(1)

    @pl.when(i == 0)
    def _project_kv():
        y = y_ref[0].astype(jnp.bfloat16)
        k = jax.lax.dot_general(y, wk_ref[...].astype(jnp.bfloat16),
                                (((1,), (0,)), ((), ())),
                                preferred_element_type=jnp.float32)
        k_sc[...] = (k + bk_ref[...]).astype(jnp.bfloat16)
        v = jax.lax.dot_general(y, wv_ref[...].astype(jnp.bfloat16),
                                (((1,), (0,)), ((), ())),
                                preferred_element_type=jnp.float32)
        v_sc[...] = (v + bv_ref[...]).astype(jnp.bfloat16)

    x = x_ref[0]
    q = jax.lax.dot_general(x.astype(jnp.bfloat16),
                            wq_ref[...].astype(jnp.bfloat16),
                            (((1,), (0,)), ((), ())),
                            preferred_element_type=jnp.float32)
    q = (q + bq_ref[...]).astype(jnp.bfloat16)
    nch = v_sc.shape[0] // _CH
    ss, ms = [], []
    for j in range(nch):
        sj = jax.lax.dot_general(q, k_sc[j * _CH:(j + 1) * _CH, :],
                                 (((1,), (1,)), ((), ())),
                                 preferred_element_type=jnp.float32)
        ss.append(sj)
        ms.append(jnp.max(sj, axis=-1, keepdims=True))
    m = ms[0]
    for mj in ms[1:]:
        m = jnp.maximum(m, mj)
    o = None
    ls = []
    for j in range(nch):
        ej = jnp.exp(ss[j] - m)
        ls.append(jnp.sum(ej, axis=-1, keepdims=True))
        oj = jax.lax.dot_general(ej.astype(jnp.bfloat16),
                                 v_sc[j * _CH:(j + 1) * _CH, :],
                                 (((1,), (0,)), ((), ())),
                                 preferred_element_type=jnp.float32)
        o = oj if o is None else o + oj
    l = ls[0]
    for lj in ls[1:]:
        l = l + lj
    o_ref[0] = o * (1.0 / l) + x


def kernel(x, y, Wq, bq, Wk, bk, Wv, bv):
    B, SX, D = x.shape
    SY = y.shape[1]
    bq2 = bq.reshape(1, D)
    bk2 = bk.reshape(1, D)
    bv2 = bv.reshape(1, D)

    return pl.pallas_call(
        _fused_kernel,
        grid=(B, SX // _BX),
        in_specs=[
            pl.BlockSpec((1, _BX, D), lambda b, i: (b, i, 0)),
            pl.BlockSpec((1, SY, D), lambda b, i: (b, 0, 0)),
            pl.BlockSpec((D, D), lambda b, i: (0, 0)),
            pl.BlockSpec((1, D), lambda b, i: (0, 0)),
            pl.BlockSpec((D, D), lambda b, i: (0, 0)),
            pl.BlockSpec((1, D), lambda b, i: (0, 0)),
            pl.BlockSpec((D, D), lambda b, i: (0, 0)),
            pl.BlockSpec((1, D), lambda b, i: (0, 0)),
        ],
        out_specs=pl.BlockSpec((1, _BX, D), lambda b, i: (b, i, 0)),
        out_shape=jax.ShapeDtypeStruct((B, SX, D), jnp.float32),
        scratch_shapes=[
            pltpu.VMEM((SY, D), jnp.bfloat16),
            pltpu.VMEM((SY, D), jnp.bfloat16),
        ],
    )(x, y, Wq, bq2, Wk, bk2, Wv, bv2)


# speedup vs baseline: 1.1770x; 1.0137x over previous
import jax
import jax.numpy as jnp
from jax.experimental import pallas as pl
from jax.experimental.pallas import tpu as pltpu

_BX = 512
_CH = 256


def _fused_kernel(x_ref, y_ref, wq_ref, bq_ref, wk_ref, bk_ref,
                  wv_ref, bv_ref, o_ref, k_sc, v_sc):
    i = pl.program_id(1)

    @pl.when(i == 0)
    def _project_kv():
        y = y_ref[0].astype(jnp.bfloat16)
        k = jax.lax.dot_general(y, wk_ref[...].astype(jnp.bfloat16),
                                (((1,), (0,)), ((), ())),
                                preferred_element_type=jnp.float32)
        k_sc[...] = (k + bk_ref[...]).astype(jnp.bfloat16)
        v = jax.lax.dot_general(y, wv_ref[...].astype(jnp.bfloat16),
                                (((1,), (0,)), ((), ())),
                                preferred_element_type=jnp.float32)
        v_sc[...] = (v + bv_ref[...]).astype(jnp.bfloat16)

    x = x_ref[0]
    q = jax.lax.dot_general(x.astype(jnp.bfloat16),
                            wq_ref[...].astype(jnp.bfloat16),
                            (((1,), (0,)), ((), ())),
                            preferred_element_type=jnp.float32)
    q = (q + bq_ref[...]).astype(jnp.bfloat16)
    nch = v_sc.shape[0] // _CH
    ss, ms = [], []
    for j in range(nch):
        sj = jax.lax.dot_general(q, k_sc[j * _CH:(j + 1) * _CH, :],
                                 (((1,), (1,)), ((), ())),
                                 preferred_element_type=jnp.float32)
        ss.append(sj)
        ms.append(jnp.max(sj, axis=-1, keepdims=True))
    m = ms[0]
    for mj in ms[1:]:
        m = jnp.maximum(m, mj)
    o = None
    ls = []
    for j in range(nch):
        ej = jnp.exp(ss[j] - m)
        ls.append(jnp.sum(ej, axis=-1, keepdims=True))
        oj = jax.lax.dot_general(ej.astype(jnp.bfloat16),
                                 v_sc[j * _CH:(j + 1) * _CH, :],
                                 (((1,), (0,)), ((), ())),
                                 preferred_element_type=jnp.float32)
        o = oj if o is None else o + oj
    l = ls[0]
    for lj in ls[1:]:
        l = l + lj
    o_ref[0] = o * (1.0 / l) + x


def kernel(x, y, Wq, bq, Wk, bk, Wv, bv):
    B, SX, D = x.shape
    SY = y.shape[1]
    bq2 = bq.reshape(1, D)
    bk2 = bk.reshape(1, D)
    bv2 = bv.reshape(1, D)

    return pl.pallas_call(
        _fused_kernel,
        grid=(B, SX // _BX),
        in_specs=[
            pl.BlockSpec((1, _BX, D), lambda b, i: (b, i, 0)),
            pl.BlockSpec((1, SY, D), lambda b, i: (b, 0, 0)),
            pl.BlockSpec((D, D), lambda b, i: (0, 0)),
            pl.BlockSpec((1, D), lambda b, i: (0, 0)),
            pl.BlockSpec((D, D), lambda b, i: (0, 0)),
            pl.BlockSpec((1, D), lambda b, i: (0, 0)),
            pl.BlockSpec((D, D), lambda b, i: (0, 0)),
            pl.BlockSpec((1, D), lambda b, i: (0, 0)),
        ],
        out_specs=pl.BlockSpec((1, _BX, D), lambda b, i: (b, i, 0)),
        out_shape=jax.ShapeDtypeStruct((B, SX, D), jnp.float32),
        scratch_shapes=[
            pltpu.VMEM((SY, D), jnp.bfloat16),
            pltpu.VMEM((SY, D), jnp.bfloat16),
        ],
    )(x, y, Wq, bq2, Wk, bk2, Wv, bv2)
